# baseline (speedup 1.0000x reference)
import functools

import jax
import jax.numpy as jnp
from jax import lax
from jax.experimental import pallas as pl
from jax.experimental.pallas import tpu as pltpu
from jax.experimental.pallas import tpu_sc as plsc

NC = 2
NS = 16
NW = NC * NS
LANES = 16
CHUNK = 80
NBUF = 4
WCH = 8
ZCHUNK = 80

_DNUMS = lax.GatherDimensionNumbers(
    offset_dims=(), collapsed_slice_dims=(0,), start_index_map=(0,))


def _mm_body(x_ref, w_ref, h_ref):
    h_ref[...] = lax.dot_general(
        x_ref[...], w_ref[...], (((1,), (1,)), ((), ())),
        preferred_element_type=jnp.float32)


def _add_body(a_ref, b_ref, o_ref):
    o_ref[...] = a_ref[0] + b_ref[0]


@functools.lru_cache(maxsize=None)
def _make_spmm(n, d, n_full):
    assert n_full % NBUF == 0 and n_full % WCH == 0 and n_full >= 2 * WCH
    assert n % ZCHUNK == 0 and d % LANES == 0
    n_row_chunks = n // ZCHUNK
    nsub = d // LANES

    mesh = plsc.VectorSubcoreMesh(core_axis_name="c", subcore_axis_name="s")

    scratch = (
        [pltpu.VMEM((2 * WCH, 1, CHUNK), jnp.int32),
         pltpu.VMEM((2 * WCH, 1, CHUNK), jnp.int32),
         pltpu.VMEM((2 * WCH, 1, CHUNK), jnp.float32)]
        + [pltpu.VMEM((CHUNK, d), jnp.float32)] * NBUF
        + [pltpu.VMEM_SHARED((n, d), jnp.float32)]
        + [pltpu.SemaphoreType.DMA] * (2 * NBUF + 1)
    )

    @functools.partial(
        pl.kernel,
        mesh=mesh,
        out_type=jax.ShapeDtypeStruct((NC, n, d), jnp.float32),
        scratch_types=scratch,
    )
    def _spmm(h_hbm, src_hbm, dst_hbm, val_hbm, out_hbm,
              srcw, dstw, valw, *rest):
        rbufs = rest[:NBUF]
        acc_sh = rest[NBUF]
        gsems = rest[NBUF + 1:2 * NBUF + 1]
        ssems = rest[2 * NBUF + 1:3 * NBUF + 1]
        isem = rest[3 * NBUF + 1]
        cid = lax.axis_index("c")
        sid = lax.axis_index("s")
        wid = sid * NC + cid
        n_my_rc = (n_row_chunks - sid + NS - 1) // NS

        def _widx(c):
            return ((c // WCH) % 2) * WCH + c % WCH

        def _g_issue(c, b):
            pltpu.async_copy(h_hbm.at[srcw.at[_widx(c), 0]], rbufs[b], gsems[b])

        def _g_wait(c, b):
            pltpu.make_async_copy(h_hbm.at[srcw.at[_widx(c), 0]], rbufs[b],
                                  gsems[b]).wait()

        def _s_issue(c, b):
            pltpu.async_copy(rbufs[b], acc_sh.at[dstw.at[_widx(c), 0]],
                             ssems[b], add=True)

        def _s_wait(c, b):
            pltpu.make_async_copy(rbufs[b], acc_sh.at[dstw.at[_widx(c), 0]],
                                  ssems[b]).wait()

        def _refill_issue(c0, slot):
            dstsl = pl.ds(slot * WCH, WCH)
            pltpu.async_copy(src_hbm.at[wid, pl.ds(c0, WCH)], srcw.at[dstsl], isem)
            pltpu.async_copy(dst_hbm.at[wid, pl.ds(c0, WCH)], dstw.at[dstsl], isem)
            pltpu.async_copy(val_hbm.at[wid, pl.ds(c0, WCH)], valw.at[dstsl], isem)

        def _refill_wait(c0, slot):
            dstsl = pl.ds(slot * WCH, WCH)
            pltpu.make_async_copy(src_hbm.at[wid, pl.ds(c0, WCH)],
                                  srcw.at[dstsl], isem).wait()
            pltpu.make_async_copy(dst_hbm.at[wid, pl.ds(c0, WCH)],
                                  dstw.at[dstsl], isem).wait()
            pltpu.make_async_copy(val_hbm.at[wid, pl.ds(c0, WCH)],
                                  valw.at[dstsl], isem).wait()

        def _scale(c, b):
            rows_ref = rbufs[b]
            wrow = _widx(c)

            @plsc.parallel_loop(0, CHUNK, 1, unroll=8)
            def _edge(j):
                base = (j >> 4) << 4
                vv = valw[wrow, 0, pl.ds(base, LANES)]
                bv = lax.gather(
                    vv, jnp.full((LANES, 1), j - base, jnp.int32), _DNUMS,
                    (1,), mode=lax.GatherScatterMode.PROMISE_IN_BOUNDS)
                for k in range(nsub):
                    sl = pl.ds(k * LANES, LANES)
                    rows_ref[j, sl] = rows_ref[j, sl] * bv

        zero16 = jnp.zeros((LANES,), jnp.float32)
        r0 = rbufs[0]

        def _zrow(r, carry):
            for k in range(nsub):
                r0[r, pl.ds(k * LANES, LANES)] = zero16
            return carry
        lax.fori_loop(0, ZCHUNK, _zrow, 0)

        def _zcp(i, carry):
            rr = (sid + i * NS) * ZCHUNK
            pltpu.sync_copy(r0.at[pl.ds(0, ZCHUNK)], acc_sh.at[pl.ds(rr, ZCHUNK)])
            return carry
        lax.fori_loop(0, n_my_rc, _zcp, 0)

        pltpu.sync_copy(src_hbm.at[wid, pl.ds(0, 2 * WCH)], srcw)
        pltpu.sync_copy(dst_hbm.at[wid, pl.ds(0, 2 * WCH)], dstw)
        pltpu.sync_copy(val_hbm.at[wid, pl.ds(0, 2 * WCH)], valw)
        for b in range(NBUF - 1):
            _g_issue(b, b)

        plsc.subcore_barrier()

        def _body(j, carry):
            for k in range(NBUF):
                c = NBUF * j + k
                _g_wait(c, k)
                _scale(c, k)
                _s_issue(c, k)
                kp = (k + NBUF - 1) % NBUF
                if k == 0:
                    @pl.when(j > 0)
                    def _():
                        _s_wait(c - 1, kp)
                    @pl.when(jnp.logical_and(
                        jnp.logical_and(c % WCH == WCH - NBUF,
                                        c > WCH - NBUF),
                        c + NBUF < n_full))
                    def _():
                        _refill_wait(c + NBUF, ((c // WCH) + 1) % 2)
                    @pl.when(jnp.logical_and(
                        jnp.logical_and(c % WCH == 0, c > 0),
                        c + WCH < n_full))
                    def _():
                        _refill_issue(c + WCH, ((c // WCH) + 1) % 2)
                else:
                    _s_wait(c - 1, kp)
                _g_issue(c + NBUF - 1, kp)
            return carry
        lax.fori_loop(0, n_full // NBUF - 1, _body, 0)

        for k in range(NBUF):
            c = n_full - NBUF + k
            _g_wait(c, k)
            _scale(c, k)
            kp = (k + NBUF - 1) % NBUF
            _s_wait(c - 1, kp)
            if k == 0:
                _g_issue(c + NBUF - 1, kp)
            _s_issue(c, k)
        _s_wait(n_full - 1, NBUF - 1)

        plsc.subcore_barrier()

        def _ocp(i, carry):
            rr = (sid + i * NS) * ZCHUNK
            pltpu.sync_copy(acc_sh.at[pl.ds(rr, ZCHUNK)],
                            out_hbm.at[cid, pl.ds(rr, ZCHUNK)])
            return carry
        lax.fori_loop(0, n_my_rc, _ocp, 0)

    return _spmm


def kernel(x, edge_index, edge_values, W):
    n, _ = x.shape
    d = W.shape[0]
    e = edge_values.shape[0]
    rb = 1000
    grid = n // rb

    h = pl.pallas_call(
        _mm_body,
        grid=(grid,),
        in_specs=[pl.BlockSpec((rb, x.shape[1]), lambda i: (i, 0)),
                  pl.BlockSpec(W.shape, lambda i: (0, 0))],
        out_specs=pl.BlockSpec((rb, d), lambda i: (i, 0)),
        out_shape=jax.ShapeDtypeStruct((n, d), jnp.float32),
    )(x, W)

    assert e % NW == 0
    ew = e // NW
    step = WCH * CHUNK
    ew_pad = -(-ew // step) * step
    n_full = ew_pad // CHUNK

    def _pack(a):
        a = a.reshape(NW, ew)
        a = jnp.pad(a, ((0, 0), (0, ew_pad - ew)))
        return a.reshape(NW, n_full, 1, CHUNK)

    src_p = _pack(edge_index[1])
    dst_p = _pack(edge_index[0])
    val_p = _pack(edge_values)

    partials = _make_spmm(n, d, n_full)(h, src_p, dst_p, val_p)

    out = pl.pallas_call(
        _add_body,
        grid=(grid,),
        in_specs=[pl.BlockSpec((1, rb, d), lambda i: (0, i, 0)),
                  pl.BlockSpec((1, rb, d), lambda i: (1, i, 0))],
        out_specs=pl.BlockSpec((rb, d), lambda i: (i, 0)),
        out_shape=jax.ShapeDtypeStruct((n, d), jnp.float32),
    )(partials, partials)
    return out


# speedup vs baseline: 1.7541x; 1.7541x over previous
import functools

import jax
import jax.numpy as jnp
from jax import lax
from jax.experimental import pallas as pl
from jax.experimental.pallas import tpu as pltpu
from jax.experimental.pallas import tpu_sc as plsc

NC = 2
NS = 16
NW = NC * NS
LANES = 16
CHUNK = 128
ZCHUNK = 80

_DNUMS = lax.GatherDimensionNumbers(
    offset_dims=(), collapsed_slice_dims=(0,), start_index_map=(0,))


def _mm_body(x_ref, w_ref, h_ref):
    h_ref[...] = lax.dot_general(
        x_ref[...], w_ref[...], (((1,), (1,)), ((), ())),
        preferred_element_type=jnp.float32)


def _add_body(a_ref, b_ref, o_ref):
    o_ref[...] = a_ref[...] + b_ref[...]


@functools.lru_cache(maxsize=None)
def _make_spmm(n, d, e):
    assert e % NW == 0
    e_per_w = e // NW
    n_full = e_per_w // CHUNK
    tail = e_per_w - n_full * CHUNK
    assert n_full >= 4 and n_full % 2 == 0
    assert e_per_w % 8 == 0 and tail % 8 == 0
    assert n % ZCHUNK == 0 and d % LANES == 0
    n_row_chunks = n // ZCHUNK
    nsub = d // LANES

    mesh = plsc.VectorSubcoreMesh(core_axis_name="c", subcore_axis_name="s")

    def _scale_rows(rows_ref, val_ref, nrows):
        def _g(g, carry):
            vv = val_ref[pl.ds(g * LANES, LANES)]
            for j in range(LANES):
                bv = lax.gather(
                    vv, jnp.full((LANES, 1), j, jnp.int32), _DNUMS, (1,),
                    mode=lax.GatherScatterMode.PROMISE_IN_BOUNDS)
                for k in range(nsub):
                    sl = pl.ds(k * LANES, LANES)
                    rows_ref[g * LANES + j, sl] = rows_ref[g * LANES + j, sl] * bv
            return carry
        lax.fori_loop(0, nrows // LANES, _g, 0)

    scratch = [
        pltpu.VMEM((CHUNK,), jnp.int32),
        pltpu.VMEM((CHUNK,), jnp.int32),
        pltpu.VMEM((CHUNK,), jnp.float32),
        pltpu.VMEM((CHUNK, d), jnp.float32),
        pltpu.VMEM((CHUNK,), jnp.int32),
        pltpu.VMEM((CHUNK,), jnp.int32),
        pltpu.VMEM((CHUNK,), jnp.float32),
        pltpu.VMEM((CHUNK, d), jnp.float32),
        pltpu.VMEM_SHARED((n, d), jnp.float32),
        pltpu.SemaphoreType.DMA,
        pltpu.SemaphoreType.DMA,
        pltpu.SemaphoreType.DMA,
        pltpu.SemaphoreType.DMA,
    ]
    if tail:
        scratch += [
            pltpu.VMEM((tail,), jnp.int32),
            pltpu.VMEM((tail,), jnp.int32),
            pltpu.VMEM((tail,), jnp.float32),
            pltpu.VMEM((tail, d), jnp.float32),
        ]

    @functools.partial(
        pl.kernel,
        mesh=mesh,
        out_type=jax.ShapeDtypeStruct((NC, n, d), jnp.float32),
        scratch_types=scratch,
    )
    def _spmm(h_hbm, dst_hbm, src_hbm, val_hbm, out_hbm,
              s0, d0, v0, r0, s1, d1, v1, r1, acc_sh, gs0, gs1, ss0, ss1,
              *tailbufs):
        cid = lax.axis_index("c")
        sid = lax.axis_index("s")
        wid = sid * NC + cid
        e0 = wid * e_per_w
        n_my_rc = (n_row_chunks - sid + NS - 1) // NS

        zero16 = jnp.zeros((LANES,), jnp.float32)

        def _zrow(r, carry):
            for k in range(nsub):
                r0[r, pl.ds(k * LANES, LANES)] = zero16
            return carry
        lax.fori_loop(0, ZCHUNK, _zrow, 0)

        def _zcp(i, carry):
            rr = (sid + i * NS) * ZCHUNK
            pltpu.sync_copy(r0.at[pl.ds(0, ZCHUNK)], acc_sh.at[pl.ds(rr, ZCHUNK)])
            return carry
        lax.fori_loop(0, n_my_rc, _zcp, 0)

        def _idxcpy(c, sv, dv, vv):
            base = e0 + c * CHUNK
            pltpu.sync_copy(src_hbm.at[pl.ds(base, CHUNK)], sv)
            pltpu.sync_copy(dst_hbm.at[pl.ds(base, CHUNK)], dv)
            pltpu.sync_copy(val_hbm.at[pl.ds(base, CHUNK)], vv)

        _idxcpy(0, s0, d0, v0)
        pltpu.async_copy(h_hbm.at[s0], r0, gs0)
        _idxcpy(1, s1, d1, v1)
        pltpu.async_copy(h_hbm.at[s1], r1, gs1)

        plsc.subcore_barrier()

        def _body(j, carry):
            pltpu.make_async_copy(h_hbm.at[s0], r0, gs0).wait()
            _scale_rows(r0, v0, CHUNK)
            pltpu.async_copy(r0, acc_sh.at[d0], ss0, add=True)
            pltpu.make_async_copy(h_hbm.at[s1], r1, gs1).wait()
            _scale_rows(r1, v1, CHUNK)
            pltpu.async_copy(r1, acc_sh.at[d1], ss1, add=True)
            pltpu.make_async_copy(r0, acc_sh.at[d0], ss0).wait()
            _idxcpy(2 * j + 2, s0, d0, v0)
            pltpu.async_copy(h_hbm.at[s0], r0, gs0)
            pltpu.make_async_copy(r1, acc_sh.at[d1], ss1).wait()
            _idxcpy(2 * j + 3, s1, d1, v1)
            pltpu.async_copy(h_hbm.at[s1], r1, gs1)
            return carry
        lax.fori_loop(0, n_full // 2 - 1, _body, 0)

        pltpu.make_async_copy(h_hbm.at[s0], r0, gs0).wait()
        _scale_rows(r0, v0, CHUNK)
        pltpu.async_copy(r0, acc_sh.at[d0], ss0, add=True)
        pltpu.make_async_copy(h_hbm.at[s1], r1, gs1).wait()
        _scale_rows(r1, v1, CHUNK)
        pltpu.async_copy(r1, acc_sh.at[d1], ss1, add=True)
        pltpu.make_async_copy(r0, acc_sh.at[d0], ss0).wait()
        pltpu.make_async_copy(r1, acc_sh.at[d1], ss1).wait()

        if tail:
            st, dt, vt, rt = tailbufs
            base = e0 + n_full * CHUNK
            pltpu.sync_copy(src_hbm.at[pl.ds(base, tail)], st)
            pltpu.sync_copy(dst_hbm.at[pl.ds(base, tail)], dt)
            pltpu.sync_copy(val_hbm.at[pl.ds(base, tail)], vt)
            pltpu.async_copy(h_hbm.at[st], rt, gs0).wait()
            _scale_rows(rt, vt, tail)
            pltpu.sync_copy(rt, acc_sh.at[dt], add=True)

        plsc.subcore_barrier()

        def _ocp(i, carry):
            rr = (sid + i * NS) * ZCHUNK
            pltpu.sync_copy(acc_sh.at[pl.ds(rr, ZCHUNK)],
                            out_hbm.at[cid, pl.ds(rr, ZCHUNK)])
            return carry
        lax.fori_loop(0, n_my_rc, _ocp, 0)

    return _spmm


def kernel(x, edge_index, edge_values, W):
    n, _ = x.shape
    d = W.shape[0]
    e = edge_values.shape[0]
    rb = 1000
    grid = n // rb

    h = pl.pallas_call(
        _mm_body,
        grid=(grid,),
        in_specs=[pl.BlockSpec((rb, x.shape[1]), lambda i: (i, 0)),
                  pl.BlockSpec(W.shape, lambda i: (0, 0))],
        out_specs=pl.BlockSpec((rb, d), lambda i: (i, 0)),
        out_shape=jax.ShapeDtypeStruct((n, d), jnp.float32),
    )(x, W)

    partials = _make_spmm(n, d, e)(h, edge_index[0], edge_index[1], edge_values)

    out = pl.pallas_call(
        _add_body,
        grid=(grid,),
        in_specs=[pl.BlockSpec((rb, d), lambda i: (i, 0)),
                  pl.BlockSpec((rb, d), lambda i: (i, 0))],
        out_specs=pl.BlockSpec((rb, d), lambda i: (i, 0)),
        out_shape=jax.ShapeDtypeStruct((n, d), jnp.float32),
    )(partials[0], partials[1])
    return out
